# baseline (speedup 1.0000x reference)
import functools

import jax
import jax.numpy as jnp
from jax import lax
from jax.experimental import pallas as pl
from jax.experimental.pallas import tpu as pltpu
from jax.experimental.pallas import tpu_sc as plsc

B = 16384
F = 26
V = 100000
D = 64

NW = 32
PLANES = F * D
PPW = PLANES // NW
QB = B // 4

_mesh = plsc.VectorSubcoreMesh(core_axis_name="c", subcore_axis_name="s")


@functools.partial(
    pl.kernel,
    mesh=_mesh,
    compiler_params=pltpu.CompilerParams(needs_layout_passes=False),
    out_type=jax.ShapeDtypeStruct((F, D, B), jnp.float32),
    scratch_types=[
        pltpu.VMEM((V,), jnp.float32),
        pltpu.VMEM((B,), jnp.int32),
        pltpu.VMEM((QB,), jnp.float32),
        pltpu.VMEM((QB,), jnp.float32),
        pltpu.SemaphoreType.DMA,
        pltpu.SemaphoreType.DMA,
        pltpu.SemaphoreType.DMA,
        pltpu.SemaphoreType.DMA,
    ],
)
def _sc_plane_gather(tt_hbm, st_hbm, out_hbm, plane, idx, ob0, ob1,
                     psem, isem, ssem0, ssem1):
    wid = lax.axis_index("s") * 2 + lax.axis_index("c")
    p0 = wid * PPW
    f0 = lax.shift_right_logical(p0, 6)
    d0 = lax.bitwise_and(p0, D - 1)

    pltpu.async_copy(st_hbm.at[f0], idx, isem)
    pltpu.async_copy(tt_hbm.at[f0, d0], plane, psem)

    def _plane(i, carry):
        p = p0 + i
        f = lax.shift_right_logical(p, 6)
        d = lax.bitwise_and(p, D - 1)

        pltpu.make_async_copy(tt_hbm.at[f, d], plane, psem).wait()

        @pl.when(jnp.logical_or(i == 0, d == 0))
        def _():
            pltpu.make_async_copy(st_hbm.at[f], idx, isem).wait()

        for q in range(4):
            ob = ob0 if q % 2 == 0 else ob1
            sem = ssem0 if q % 2 == 0 else ssem1
            drain = pltpu.make_async_copy(
                ob, out_hbm.at[f, d, pl.ds(q * QB, QB)], sem)
            if q < 2:
                @pl.when(i > 0)
                def _():
                    drain.wait()
            else:
                drain.wait()

            @plsc.parallel_loop(0, QB // 16, unroll=16)
            def _vec(g):
                iv = idx[pl.ds(q * QB + g * 16, 16)]
                ob[pl.ds(g * 16, 16)] = plsc.load_gather(plane, [iv])

            if q == 3:
                @pl.when(i + 1 < PPW)
                def _():
                    pn = p + 1
                    fn = lax.shift_right_logical(pn, 6)
                    dn = lax.bitwise_and(pn, D - 1)
                    pltpu.async_copy(tt_hbm.at[fn, dn], plane, psem)

                    @pl.when(dn == 0)
                    def _():
                        pltpu.async_copy(st_hbm.at[fn], idx, isem)

            pltpu.async_copy(ob, out_hbm.at[f, d, pl.ds(q * QB, QB)], sem)
        return carry

    lax.fori_loop(0, PPW, _plane, 0)
    pltpu.make_async_copy(ob0, out_hbm.at[0, 0, pl.ds(0, QB)], ssem0).wait()
    pltpu.make_async_copy(ob1, out_hbm.at[0, 0, pl.ds(0, QB)], ssem1).wait()


def kernel(tables, s):
    tt = tables.transpose(0, 2, 1)
    st = s.T
    o = _sc_plane_gather(tt, st)
    return o.transpose(2, 0, 1)
